# Initial kernel scaffold; baseline (speedup 1.0000x reference)
import jax
import jax.numpy as jnp
from jax.experimental import pallas as pl


def kernel(x, W1, b1):
    raise NotImplementedError("write your pallas kernel here")


# speedup vs baseline: 2.8061x; 2.8061x over previous
import functools

import jax
import jax.numpy as jnp
from jax import lax
from jax.experimental import pallas as pl
from jax.experimental.pallas import tpu as pltpu
from jax.experimental.pallas import tpu_sc as plsc

N = 8192
D = 512
H = 256
K = 8

ROW_BLK = 256
COL_BLK = 512


def _feat_kernel(x_ref, w_ref, b_ref, hn_ref, hnt_ref):
    h = lax.dot_general(
        x_ref[...], w_ref[...], (((1,), (0,)), ((), ())),
        preferred_element_type=jnp.float32,
        precision=lax.Precision.DEFAULT,
    )
    h = h + b_ref[...]
    h = jnp.where(h >= 0, h, 0.01 * h)
    ssq = jnp.sum(h * h, axis=1, keepdims=True)
    hn = h / (jnp.sqrt(ssq) + 1e-8)
    hn_ref[...] = hn
    hnt_ref[...] = hn.T


def _topk_kernel(a_ref, ht_ref, nbr_ref, rval_ref, ridx_ref):
    j = pl.program_id(1)

    @pl.when(j == 0)
    def _init():
        rval_ref[...] = jnp.full((ROW_BLK, K), -jnp.inf, jnp.float32)
        ridx_ref[...] = jnp.zeros((ROW_BLK, K), jnp.int32)

    s = lax.dot_general(
        a_ref[...], ht_ref[...], (((1,), (0,)), ((), ())),
        preferred_element_type=jnp.float32,
        precision=lax.Precision.DEFAULT,
    )
    iota = lax.broadcasted_iota(jnp.int32, (ROW_BLK, COL_BLK), 1) + j * COL_BLK

    bvals, bidxs = [], []
    for t in range(K):
        m = jnp.max(s, axis=1, keepdims=True)
        eq = s == m
        idx = jnp.min(jnp.where(eq, iota, jnp.int32(N)), axis=1, keepdims=True)
        bvals.append(m)
        bidxs.append(idx)
        s = jnp.where(eq, -jnp.inf, s)

    cv = jnp.concatenate([rval_ref[...]] + bvals, axis=1)
    ci = jnp.concatenate([ridx_ref[...]] + bidxs, axis=1)
    nv, ni = [], []
    for t in range(K):
        m = jnp.max(cv, axis=1, keepdims=True)
        eq = cv == m
        idx = jnp.min(jnp.where(eq, ci, jnp.int32(N)), axis=1, keepdims=True)
        nv.append(m)
        ni.append(idx)
        cv = jnp.where(eq & (ci == idx), -jnp.inf, cv)
    rval_ref[...] = jnp.concatenate(nv, axis=1)
    ridx_ref[...] = jnp.concatenate(ni, axis=1)

    @pl.when(j == pl.num_programs(1) - 1)
    def _emit():
        nbr_ref[...] = ridx_ref[...]


def _knn_neighbors(x, W1, b1):
    hn, hnt = pl.pallas_call(
        _feat_kernel,
        grid=(N // 512,),
        in_specs=[
            pl.BlockSpec((512, D), lambda i: (i, 0)),
            pl.BlockSpec((D, H), lambda i: (0, 0)),
            pl.BlockSpec((1, H), lambda i: (0, 0)),
        ],
        out_specs=[
            pl.BlockSpec((512, H), lambda i: (i, 0)),
            pl.BlockSpec((H, 512), lambda i: (0, i)),
        ],
        out_shape=[
            jax.ShapeDtypeStruct((N, H), jnp.float32),
            jax.ShapeDtypeStruct((H, N), jnp.float32),
        ],
    )(x, W1, b1.reshape(1, H))

    nbr = pl.pallas_call(
        _topk_kernel,
        grid=(N // ROW_BLK, N // COL_BLK),
        in_specs=[
            pl.BlockSpec((ROW_BLK, H), lambda i, j: (i, 0)),
            pl.BlockSpec((H, COL_BLK), lambda i, j: (0, j)),
        ],
        out_specs=pl.BlockSpec((ROW_BLK, K), lambda i, j: (i, 0)),
        out_shape=jax.ShapeDtypeStruct((N, K), jnp.int32),
        scratch_shapes=[
            pltpu.VMEM((ROW_BLK, K), jnp.float32),
            pltpu.VMEM((ROW_BLK, K), jnp.int32),
        ],
    )(hn, hnt)
    return nbr


def _make_gather_mean():
    info = plsc.get_sparse_core_info()
    nw = info.num_cores * info.num_subcores
    q_per_w = N // nw
    qc = 8
    rows_per_chunk = qc * K
    n_chunks = q_per_w // qc
    mesh = plsc.VectorSubcoreMesh(core_axis_name="c", subcore_axis_name="s")

    @functools.partial(
        pl.kernel,
        mesh=mesh,
        out_type=jax.ShapeDtypeStruct((N, D), jnp.float32),
        scratch_types=[
            pltpu.VMEM((rows_per_chunk,), jnp.int32),
            pltpu.VMEM((rows_per_chunk, D), jnp.float32),
            pltpu.VMEM((qc, D), jnp.float32),
            pltpu.SemaphoreType.DMA,
        ],
    )
    def gather_mean(x_hbm, idx_hbm, out_hbm, idx_v, rows_v, acc_v, sem):
        wid = lax.axis_index("s") * info.num_cores + lax.axis_index("c")
        qbase = wid * q_per_w

        def chunk_body(c, _):
            pltpu.sync_copy(
                idx_hbm.at[pl.ds((qbase + c * qc) * K, rows_per_chunk)], idx_v)
            pltpu.async_copy(x_hbm.at[idx_v], rows_v, sem).wait()

            def q_body(q, _):
                def g_body(g, _):
                    col = pl.ds(g * 16, 16)
                    acc = rows_v[q * K, col]
                    for r in range(1, K):
                        acc = acc + rows_v[q * K + r, col]
                    acc_v[q, col] = acc * 0.125
                    return 0
                return lax.fori_loop(0, D // 16, g_body, 0)

            lax.fori_loop(0, qc, q_body, 0)
            pltpu.sync_copy(acc_v, out_hbm.at[pl.ds(qbase + c * qc, qc)])
            return 0

        lax.fori_loop(0, n_chunks, chunk_body, 0)

    return gather_mean


def kernel(x, W1, b1):
    nbr = _knn_neighbors(x, W1, b1)
    row = nbr.reshape(-1)
    edge_attr = _make_gather_mean()(x, row)
    col = jnp.repeat(jnp.arange(N, dtype=jnp.int32), K)
    edge_index = jnp.stack([row, col], axis=0)
    return (x, edge_index, edge_attr)


# speedup vs baseline: 3.9534x; 1.4089x over previous
import functools

import jax
import jax.numpy as jnp
from jax import lax
from jax.experimental import pallas as pl
from jax.experimental.pallas import tpu as pltpu
from jax.experimental.pallas import tpu_sc as plsc

N = 8192
D = 512
H = 256
K = 8

ROW_BLK = 256
COL_BLK = 512


def _feat_kernel(x_ref, w_ref, b_ref, hn_ref, hnt_ref):
    h = lax.dot_general(
        x_ref[...], w_ref[...], (((1,), (0,)), ((), ())),
        preferred_element_type=jnp.float32,
        precision=lax.Precision.DEFAULT,
    )
    h = h + b_ref[...]
    h = jnp.where(h >= 0, h, 0.01 * h)
    ssq = jnp.sum(h * h, axis=1, keepdims=True)
    hn = h / (jnp.sqrt(ssq) + 1e-8)
    hn_ref[...] = hn
    hnt_ref[...] = hn.T


def _topk_kernel(a_ref, ht_ref, nbr_ref, rval_ref, ridx_ref):
    j = pl.program_id(1)

    @pl.when(j == 0)
    def _init():
        rval_ref[...] = jnp.full((ROW_BLK, K), -jnp.inf, jnp.float32)
        ridx_ref[...] = jnp.zeros((ROW_BLK, K), jnp.float32)

    s = lax.dot_general(
        a_ref[...], ht_ref[...], (((1,), (0,)), ((), ())),
        preferred_element_type=jnp.float32,
        precision=lax.Precision.DEFAULT,
    )
    iota_row = (lax.broadcasted_iota(jnp.int32, (ROW_BLK, COL_BLK), 1)
                + j * COL_BLK).astype(jnp.float32)

    bvals, bidxs = [], []
    for t in range(K):
        m = jnp.max(s, axis=1, keepdims=True)
        eq = s == m
        idx = jnp.min(jnp.where(eq, iota_row, jnp.inf), axis=1, keepdims=True)
        bvals.append(m)
        bidxs.append(idx)
        s = jnp.where(eq, -jnp.inf, s)

    cv = jnp.concatenate([rval_ref[...]] + bvals, axis=1)
    ci = jnp.concatenate([ridx_ref[...]] + bidxs, axis=1)
    nv, ni = [], []
    for t in range(K):
        m = jnp.max(cv, axis=1, keepdims=True)
        eq = cv == m
        idx = jnp.min(jnp.where(eq, ci, jnp.float32(2 * N)), axis=1, keepdims=True)
        nv.append(m)
        ni.append(idx)
        cv = jnp.where(eq & (ci == idx), -jnp.inf, cv)
    rval_ref[...] = jnp.concatenate(nv, axis=1)
    ridx_ref[...] = jnp.concatenate(ni, axis=1)

    @pl.when(j == pl.num_programs(1) - 1)
    def _emit():
        nbr_ref[...] = jnp.clip(ridx_ref[...], 0.0, float(N - 1)).astype(jnp.int32)


def _knn_neighbors(x, W1, b1):
    hn, hnt = pl.pallas_call(
        _feat_kernel,
        grid=(N // 512,),
        in_specs=[
            pl.BlockSpec((512, D), lambda i: (i, 0)),
            pl.BlockSpec((D, H), lambda i: (0, 0)),
            pl.BlockSpec((1, H), lambda i: (0, 0)),
        ],
        out_specs=[
            pl.BlockSpec((512, H), lambda i: (i, 0)),
            pl.BlockSpec((H, 512), lambda i: (0, i)),
        ],
        out_shape=[
            jax.ShapeDtypeStruct((N, H), jnp.float32),
            jax.ShapeDtypeStruct((H, N), jnp.float32),
        ],
    )(x, W1, b1.reshape(1, H))

    nbr = pl.pallas_call(
        _topk_kernel,
        grid=(N // ROW_BLK, N // COL_BLK),
        in_specs=[
            pl.BlockSpec((ROW_BLK, H), lambda i, j: (i, 0)),
            pl.BlockSpec((H, COL_BLK), lambda i, j: (0, j)),
        ],
        out_specs=pl.BlockSpec((ROW_BLK, K), lambda i, j: (i, 0)),
        out_shape=jax.ShapeDtypeStruct((N, K), jnp.int32),
        scratch_shapes=[
            pltpu.VMEM((ROW_BLK, K), jnp.float32),
            pltpu.VMEM((ROW_BLK, K), jnp.float32),
        ],
    )(hn, hnt)
    return nbr


def _make_gather_mean():
    info = plsc.get_sparse_core_info()
    nw = info.num_cores * info.num_subcores
    q_per_w = N // nw
    qc = 8
    rows_per_chunk = qc * K
    n_chunks = q_per_w // qc
    mesh = plsc.VectorSubcoreMesh(core_axis_name="c", subcore_axis_name="s")

    @functools.partial(
        pl.kernel,
        mesh=mesh,
        out_type=jax.ShapeDtypeStruct((N, D), jnp.float32),
        scratch_types=[
            pltpu.VMEM((rows_per_chunk,), jnp.int32),
            pltpu.VMEM((rows_per_chunk, D), jnp.float32),
            pltpu.VMEM((qc, D), jnp.float32),
            pltpu.SemaphoreType.DMA,
        ],
    )
    def gather_mean(x_hbm, idx_hbm, out_hbm, idx_v, rows_v, acc_v, sem):
        wid = lax.axis_index("s") * info.num_cores + lax.axis_index("c")
        qbase = wid * q_per_w

        def chunk_body(c, _):
            pltpu.sync_copy(
                idx_hbm.at[pl.ds((qbase + c * qc) * K, rows_per_chunk)], idx_v)
            pltpu.async_copy(x_hbm.at[idx_v], rows_v, sem).wait()

            def q_body(q, _):
                def g_body(g, _):
                    col = pl.ds(g * 16, 16)
                    acc = rows_v[q * K, col]
                    for r in range(1, K):
                        acc = acc + rows_v[q * K + r, col]
                    acc_v[q, col] = acc * 0.125
                    return 0
                return lax.fori_loop(0, D // 16, g_body, 0)

            lax.fori_loop(0, qc, q_body, 0)
            pltpu.sync_copy(acc_v, out_hbm.at[pl.ds(qbase + c * qc, qc)])
            return 0

        lax.fori_loop(0, n_chunks, chunk_body, 0)

    return gather_mean


def kernel(x, W1, b1):
    nbr = _knn_neighbors(x, W1, b1)
    row = nbr.reshape(-1)
    edge_attr = _make_gather_mean()(x, row)
    col = jnp.repeat(jnp.arange(N, dtype=jnp.int32), K)
    edge_index = jnp.stack([row, col], axis=0)
    return (x, edge_index, edge_attr)


# speedup vs baseline: 4.1619x; 1.0527x over previous
import functools

import jax
import jax.numpy as jnp
from jax import lax
from jax.experimental import pallas as pl
from jax.experimental.pallas import tpu as pltpu
from jax.experimental.pallas import tpu_sc as plsc

N = 8192
D = 512
H = 256
K = 8

ROW_BLK = 256
COL_BLK = 512


def _feat_kernel(x_ref, w_ref, b_ref, hn_ref, hnt_ref):
    h = lax.dot_general(
        x_ref[...], w_ref[...], (((1,), (0,)), ((), ())),
        preferred_element_type=jnp.float32,
        precision=lax.Precision.DEFAULT,
    )
    h = h + b_ref[...]
    h = jnp.where(h >= 0, h, 0.01 * h)
    ssq = jnp.sum(h * h, axis=1, keepdims=True)
    hn = h / (jnp.sqrt(ssq) + 1e-8)
    hn_ref[...] = hn
    hnt_ref[...] = hn.T


def _topk_kernel(a_ref, ht_ref, nbr_ref, rval_ref, ridx_ref):
    j = pl.program_id(1)

    @pl.when(j == 0)
    def _init():
        rval_ref[...] = jnp.full((ROW_BLK, K), -jnp.inf, jnp.float32)
        ridx_ref[...] = jnp.zeros((ROW_BLK, K), jnp.float32)

    s = lax.dot_general(
        a_ref[...], ht_ref[...], (((1,), (0,)), ((), ())),
        preferred_element_type=jnp.float32,
        precision=lax.Precision.DEFAULT,
    )
    iota_row = (lax.broadcasted_iota(jnp.int32, (ROW_BLK, COL_BLK), 1)
                + j * COL_BLK).astype(jnp.float32)

    bvals, bidxs = [], []
    for t in range(K):
        m = jnp.max(s, axis=1, keepdims=True)
        eq = s == m
        idx = jnp.min(jnp.where(eq, iota_row, jnp.inf), axis=1, keepdims=True)
        bvals.append(m)
        bidxs.append(idx)
        s = jnp.where(eq, -jnp.inf, s)

    cv = jnp.concatenate([rval_ref[...]] + bvals, axis=1)
    ci = jnp.concatenate([ridx_ref[...]] + bidxs, axis=1)
    nv, ni = [], []
    for t in range(K):
        m = jnp.max(cv, axis=1, keepdims=True)
        eq = cv == m
        idx = jnp.min(jnp.where(eq, ci, jnp.float32(2 * N)), axis=1, keepdims=True)
        nv.append(m)
        ni.append(idx)
        cv = jnp.where(eq & (ci == idx), -jnp.inf, cv)
    rval_ref[...] = jnp.concatenate(nv, axis=1)
    ridx_ref[...] = jnp.concatenate(ni, axis=1)

    @pl.when(j == pl.num_programs(1) - 1)
    def _emit():
        nbr_ref[...] = jnp.clip(ridx_ref[...], 0.0, float(N - 1)).astype(jnp.int32)


def _features(x, W1, b1):
    return pl.pallas_call(
        _feat_kernel,
        grid=(N // 512,),
        in_specs=[
            pl.BlockSpec((512, D), lambda i: (i, 0)),
            pl.BlockSpec((D, H), lambda i: (0, 0)),
            pl.BlockSpec((1, H), lambda i: (0, 0)),
        ],
        out_specs=[
            pl.BlockSpec((512, H), lambda i: (i, 0)),
            pl.BlockSpec((H, 512), lambda i: (0, i)),
        ],
        out_shape=[
            jax.ShapeDtypeStruct((N, H), jnp.float32),
            jax.ShapeDtypeStruct((H, N), jnp.float32),
        ],
    )(x, W1, b1.reshape(1, H))


def _topk_rows(hn_rows, hnt):
    n_rows = hn_rows.shape[0]
    return pl.pallas_call(
        _topk_kernel,
        grid=(n_rows // ROW_BLK, N // COL_BLK),
        in_specs=[
            pl.BlockSpec((ROW_BLK, H), lambda i, j: (i, 0)),
            pl.BlockSpec((H, COL_BLK), lambda i, j: (0, j)),
        ],
        out_specs=pl.BlockSpec((ROW_BLK, K), lambda i, j: (i, 0)),
        out_shape=jax.ShapeDtypeStruct((n_rows, K), jnp.int32),
        scratch_shapes=[
            pltpu.VMEM((ROW_BLK, K), jnp.float32),
            pltpu.VMEM((ROW_BLK, K), jnp.float32),
        ],
    )(hn_rows, hnt)


def _make_gather_mean(n_q):
    info = plsc.get_sparse_core_info()
    nw = info.num_cores * info.num_subcores
    q_per_w = n_q // nw
    qc = 8
    rows_per_chunk = qc * K
    n_chunks = q_per_w // qc
    mesh = plsc.VectorSubcoreMesh(core_axis_name="c", subcore_axis_name="s")

    @functools.partial(
        pl.kernel,
        mesh=mesh,
        out_type=jax.ShapeDtypeStruct((n_q, D), jnp.float32),
        scratch_types=[
            pltpu.VMEM((rows_per_chunk,), jnp.int32),
            pltpu.VMEM((rows_per_chunk, D), jnp.float32),
            pltpu.VMEM((qc, D), jnp.float32),
            pltpu.SemaphoreType.DMA,
        ],
    )
    def gather_mean(x_hbm, idx_hbm, out_hbm, idx_v, rows_v, acc_v, sem):
        wid = lax.axis_index("s") * info.num_cores + lax.axis_index("c")
        qbase = wid * q_per_w

        def chunk_body(c, _):
            pltpu.sync_copy(
                idx_hbm.at[pl.ds((qbase + c * qc) * K, rows_per_chunk)], idx_v)
            pltpu.async_copy(x_hbm.at[idx_v], rows_v, sem).wait()

            def q_body(q, _):
                def g_body(g, _):
                    col = pl.ds(g * 16, 16)
                    acc = rows_v[q * K, col]
                    for r in range(1, K):
                        acc = acc + rows_v[q * K + r, col]
                    acc_v[q, col] = acc * 0.125
                    return 0
                return lax.fori_loop(0, D // 16, g_body, 0)

            lax.fori_loop(0, qc, q_body, 0)
            pltpu.sync_copy(acc_v, out_hbm.at[pl.ds(qbase + c * qc, qc)])
            return 0

        lax.fori_loop(0, n_chunks, chunk_body, 0)

    return gather_mean


def kernel(x, W1, b1):
    hn, hnt = _features(x, W1, b1)
    half = N // 2
    gather = _make_gather_mean(half)
    nbr0 = _topk_rows(hn[:half], hnt)
    ea0 = gather(x, nbr0.reshape(-1))
    nbr1 = _topk_rows(hn[half:], hnt)
    ea1 = gather(x, nbr1.reshape(-1))
    row = jnp.concatenate([nbr0.reshape(-1), nbr1.reshape(-1)])
    edge_attr = jnp.concatenate([ea0, ea1], axis=0)
    col = jnp.repeat(jnp.arange(N, dtype=jnp.int32), K)
    edge_index = jnp.stack([row, col], axis=0)
    return (x, edge_index, edge_attr)


# speedup vs baseline: 8.7739x; 2.1082x over previous
import functools

import jax
import jax.numpy as jnp
from jax import lax
from jax.experimental import pallas as pl
from jax.experimental.pallas import tpu as pltpu
from jax.experimental.pallas import tpu_sc as plsc

N = 8192
D = 512
H = 256
K = 8

ROW_BLK = 256


def _feat_kernel(x_ref, w_ref, b_ref, hn_ref, hnt_ref):
    h = lax.dot_general(
        x_ref[...], w_ref[...], (((1,), (0,)), ((), ())),
        preferred_element_type=jnp.float32,
        precision=lax.Precision.DEFAULT,
    )
    h = h + b_ref[...]
    h = jnp.where(h >= 0, h, 0.01 * h)
    ssq = jnp.sum(h * h, axis=1, keepdims=True)
    hn = h / (jnp.sqrt(ssq) + 1e-8)
    hn_ref[...] = hn
    hnt_ref[...] = hn.T


def _topk_kernel(a_ref, ht_ref, nbr_ref):
    a = a_ref[...]
    s = jnp.concatenate(
        [
            lax.dot_general(
                a, ht_ref[:, c * 512:(c + 1) * 512], (((1,), (0,)), ((), ())),
                preferred_element_type=jnp.float32,
                precision=lax.Precision.DEFAULT,
            )
            for c in range(N // 512)
        ],
        axis=1,
    )
    iota_row = lax.broadcasted_iota(jnp.int32, (ROW_BLK, N), 1).astype(
        jnp.float32)

    idxs = []
    for t in range(K):
        m = jnp.max(s, axis=1, keepdims=True)
        eq = s == m
        idx = jnp.min(jnp.where(eq, iota_row, jnp.inf), axis=1, keepdims=True)
        idxs.append(idx)
        if t < K - 1:
            s = jnp.where(eq, -jnp.inf, s)
    nbr = jnp.concatenate(idxs, axis=1)
    nbr_ref[...] = jnp.clip(nbr, 0.0, float(N - 1)).astype(jnp.int32)


def _features(x, W1, b1):
    return pl.pallas_call(
        _feat_kernel,
        grid=(N // 512,),
        in_specs=[
            pl.BlockSpec((512, D), lambda i: (i, 0)),
            pl.BlockSpec((D, H), lambda i: (0, 0)),
            pl.BlockSpec((1, H), lambda i: (0, 0)),
        ],
        out_specs=[
            pl.BlockSpec((512, H), lambda i: (i, 0)),
            pl.BlockSpec((H, 512), lambda i: (0, i)),
        ],
        out_shape=[
            jax.ShapeDtypeStruct((N, H), jnp.float32),
            jax.ShapeDtypeStruct((H, N), jnp.float32),
        ],
    )(x, W1, b1.reshape(1, H))


def _topk_rows(hn_rows, hnt):
    n_rows = hn_rows.shape[0]
    return pl.pallas_call(
        _topk_kernel,
        grid=(n_rows // ROW_BLK,),
        in_specs=[
            pl.BlockSpec((ROW_BLK, H), lambda i: (i, 0)),
            pl.BlockSpec((H, N), lambda i: (0, 0)),
        ],
        out_specs=pl.BlockSpec((ROW_BLK, K), lambda i: (i, 0)),
        out_shape=jax.ShapeDtypeStruct((n_rows, K), jnp.int32),
    )(hn_rows, hnt)


def _make_gather_mean(n_q):
    info = plsc.get_sparse_core_info()
    nw = info.num_cores * info.num_subcores
    q_per_w = n_q // nw
    qc = 8
    rows_per_chunk = qc * K
    n_chunks = q_per_w // qc
    mesh = plsc.VectorSubcoreMesh(core_axis_name="c", subcore_axis_name="s")

    @functools.partial(
        pl.kernel,
        mesh=mesh,
        out_type=jax.ShapeDtypeStruct((n_q, D), jnp.float32),
        scratch_types=[
            pltpu.VMEM((rows_per_chunk,), jnp.int32),
            pltpu.VMEM((rows_per_chunk, D), jnp.float32),
            pltpu.VMEM((qc, D), jnp.float32),
            pltpu.SemaphoreType.DMA,
        ],
    )
    def gather_mean(x_hbm, idx_hbm, out_hbm, idx_v, rows_v, acc_v, sem):
        wid = lax.axis_index("s") * info.num_cores + lax.axis_index("c")
        qbase = wid * q_per_w

        def chunk_body(c, _):
            pltpu.sync_copy(
                idx_hbm.at[pl.ds((qbase + c * qc) * K, rows_per_chunk)], idx_v)
            pltpu.async_copy(x_hbm.at[idx_v], rows_v, sem).wait()

            def q_body(q, _):
                def g_body(g, _):
                    col = pl.ds(g * 16, 16)
                    acc = rows_v[q * K, col]
                    for r in range(1, K):
                        acc = acc + rows_v[q * K + r, col]
                    acc_v[q, col] = acc * 0.125
                    return 0
                return lax.fori_loop(0, D // 16, g_body, 0)

            lax.fori_loop(0, qc, q_body, 0)
            pltpu.sync_copy(acc_v, out_hbm.at[pl.ds(qbase + c * qc, qc)])
            return 0

        lax.fori_loop(0, n_chunks, chunk_body, 0)

    return gather_mean


def kernel(x, W1, b1):
    hn, hnt = _features(x, W1, b1)
    half = N // 2
    gather = _make_gather_mean(half)
    nbr0 = _topk_rows(hn[:half], hnt)
    ea0 = gather(x, nbr0.reshape(-1))
    nbr1 = _topk_rows(hn[half:], hnt)
    ea1 = gather(x, nbr1.reshape(-1))
    row = jnp.concatenate([nbr0.reshape(-1), nbr1.reshape(-1)])
    edge_attr = jnp.concatenate([ea0, ea1], axis=0)
    col = jnp.repeat(jnp.arange(N, dtype=jnp.int32), K)
    edge_index = jnp.stack([row, col], axis=0)
    return (x, edge_index, edge_attr)
